# Initial kernel scaffold; baseline (speedup 1.0000x reference)
import jax
import jax.numpy as jnp
from jax.experimental import pallas as pl


def kernel(seq_output, hidden, con_hidden, index, cross_lengths, lengths, node_lengths, max_length, emb, W_pre, b_pre, W_q, W_v):
    raise NotImplementedError("write your pallas kernel here")


# speedup vs baseline: 19.1553x; 19.1553x over previous
import functools

import jax
import jax.numpy as jnp
from jax import lax
from jax.experimental import pallas as pl
from jax.experimental.pallas import tpu as pltpu
from jax.experimental.pallas import tpu_sc as plsc

_B, _C, _NN, _W = 8, 4, 32, 128
_DM, _DK = 512, 64
_PAD = 0

_NC, _NS = 2, 16
_NW = _NC * _NS

_TOT = _B * _NN * _W
_RPW = _TOT // _NW
_CH = 128
_NCHUNK = _RPW // _CH

_NB = 8
_BR = _NB * _W


def _sc_gather(emb, idx_flat):
    mesh = plsc.VectorSubcoreMesh(core_axis_name="c", subcore_axis_name="s")

    @functools.partial(
        pl.kernel,
        mesh=mesh,
        out_type=jax.ShapeDtypeStruct((_TOT, _DM), jnp.float32),
        scratch_types=[
            pltpu.VMEM((_RPW,), jnp.int32),
            pltpu.VMEM((_CH, _DM), jnp.float32),
            pltpu.SemaphoreType.DMA,
        ],
    )
    def gather_k(table_hbm, idx_hbm, out_hbm, idx_v, buf, sem):
        wid = lax.axis_index("s") * _NC + lax.axis_index("c")
        base = wid * _RPW
        pltpu.sync_copy(idx_hbm.at[pl.ds(base, _RPW)], idx_v)
        for i in range(_NCHUNK):
            pltpu.async_copy(
                table_hbm.at[idx_v.at[pl.ds(i * _CH, _CH)]], buf, sem
            ).wait()
            pltpu.sync_copy(buf, out_hbm.at[pl.ds(base + i * _CH, _CH)])

    return gather_k(emb, idx_flat)


def _attn_body(bags_ref, idx_ref, h_ref, wpre_ref, bpre_ref, wq_ref, wv_ref,
               out_ref):
    X = bags_ref[...]
    m = (idx_ref[...] != _PAD).astype(jnp.float32)
    Xm = X * m
    pre = jnp.dot(Xm, wpre_ref[...],
                  preferred_element_type=jnp.float32) + bpre_ref[...]
    q = jnp.dot(h_ref[0], wq_ref[...],
                preferred_element_type=jnp.float32)
    t = jnp.tanh(pre + q)
    e = jnp.dot(t, wv_ref[...],
                preferred_element_type=jnp.float32)
    p = jnp.exp(e - jnp.max(e))
    jrow = lax.broadcasted_iota(jnp.int32, (_NB, _BR), 0)
    rrow = lax.broadcasted_iota(jnp.int32, (_NB, _BR), 1)
    ohT = (rrow // _W == jrow).astype(jnp.float32)
    ssum = jnp.dot(ohT, p, preferred_element_type=jnp.float32)
    rcol = lax.broadcasted_iota(jnp.int32, (_BR, _NB), 0)
    jcol = lax.broadcasted_iota(jnp.int32, (_BR, _NB), 1)
    oh = (rcol // _W == jcol).astype(jnp.float32)
    persum = jnp.dot(oh, ssum, preferred_element_type=jnp.float32)
    ctx = jnp.dot(ohT, Xm * (p / persum),
                  preferred_element_type=jnp.float32)
    out_ref[...] = ctx


def _attn_call(bags, idx2, h, W_pre, b_pre2, W_q, W_v, interpret=False):
    grid = (_B * _NN // _NB,)
    return pl.pallas_call(
        _attn_body,
        grid=grid,
        in_specs=[
            pl.BlockSpec((_BR, _DM), lambda s: (s, 0)),
            pl.BlockSpec((_BR, 1), lambda s: (s, 0)),
            pl.BlockSpec((1, 1, _DM), lambda s: (s // (_NN // _NB), 0, 0)),
            pl.BlockSpec((_DM, _DK), lambda s: (0, 0)),
            pl.BlockSpec((1, _DK), lambda s: (0, 0)),
            pl.BlockSpec((_DM, _DK), lambda s: (0, 0)),
            pl.BlockSpec((_DK, 1), lambda s: (0, 0)),
        ],
        out_specs=pl.BlockSpec((_NB, _DM), lambda s: (s, 0)),
        out_shape=jax.ShapeDtypeStruct((_B * _NN, _DM), jnp.float32),
        interpret=interpret,
    )(bags, idx2, h.reshape(_B, 1, _DM), W_pre, b_pre2, W_q, W_v)


def kernel(seq_output, hidden, con_hidden, index, cross_lengths, lengths,
           node_lengths, max_length, emb, W_pre, b_pre, W_q, W_v):
    h = jnp.transpose(con_hidden, (1, 0, 2)).reshape(con_hidden.shape[1], -1)
    idx_flat = index[:, -1].astype(jnp.int32).reshape(_TOT)
    bags = _sc_gather(emb, idx_flat)
    ctx = _attn_call(bags, idx_flat.reshape(_TOT, 1), h, W_pre,
                     b_pre.reshape(1, _DK), W_q, W_v)
    context = jnp.broadcast_to(
        ctx.reshape(_B, 1, _NN, _DM), (_B, _C, _NN, _DM))
    return context, h


# speedup vs baseline: 22.1518x; 1.1564x over previous
import functools

import jax
import jax.numpy as jnp
from jax import lax
from jax.experimental import pallas as pl
from jax.experimental.pallas import tpu as pltpu
from jax.experimental.pallas import tpu_sc as plsc

_B, _C, _NN, _W = 8, 4, 32, 128
_DM, _DK = 512, 64
_PAD = 0

_NC, _NS = 2, 16
_NW = _NC * _NS

_TOT = _B * _NN * _W
_RPW = _TOT // _NW
_CH = 64
_NCHUNK = _RPW // _CH

_NB = 16
_BR = _NB * _W


def _sc_gather(emb, idx_flat):
    mesh = plsc.VectorSubcoreMesh(core_axis_name="c", subcore_axis_name="s")

    @functools.partial(
        pl.kernel,
        mesh=mesh,
        out_type=jax.ShapeDtypeStruct((_TOT, _DM), jnp.float32),
        scratch_types=[
            pltpu.VMEM((_RPW,), jnp.int32),
            pltpu.VMEM((2, _CH, _DM), jnp.float32),
            pltpu.SemaphoreType.DMA,
            pltpu.SemaphoreType.DMA,
        ],
    )
    def gather_k(table_hbm, idx_hbm, out_hbm, idx_v, bufs, gsem, ssem):
        wid = lax.axis_index("s") * _NC + lax.axis_index("c")
        base = wid * _RPW
        pltpu.sync_copy(idx_hbm.at[pl.ds(base, _RPW)], idx_v)
        gathers = [None] * _NCHUNK
        scatters = [None] * _NCHUNK
        gathers[0] = pltpu.async_copy(
            table_hbm.at[idx_v.at[pl.ds(0, _CH)]], bufs.at[0], gsem)
        for i in range(_NCHUNK):
            gathers[i].wait()
            if i >= 1:
                scatters[i - 1].wait()
            if i + 1 < _NCHUNK:
                gathers[i + 1] = pltpu.async_copy(
                    table_hbm.at[idx_v.at[pl.ds((i + 1) * _CH, _CH)]],
                    bufs.at[(i + 1) % 2], gsem)
            scatters[i] = pltpu.async_copy(
                bufs.at[i % 2], out_hbm.at[pl.ds(base + i * _CH, _CH)], ssem)
        scatters[_NCHUNK - 1].wait()

    return gather_k(emb, idx_flat)


def _attn_body(bags_ref, idx_ref, h_ref, wpre_ref, bpre_ref, wq_ref, wv_ref,
               out_ref):
    X = bags_ref[...]
    m = (idx_ref[...] != _PAD)
    q = jnp.dot(h_ref[0], wq_ref[...],
                preferred_element_type=jnp.float32)
    pre = jnp.where(m, jnp.dot(X, wpre_ref[...],
                               preferred_element_type=jnp.float32), 0.0)
    t = jnp.tanh(pre + bpre_ref[...] + q)
    e = jnp.dot(t, wv_ref[...],
                preferred_element_type=jnp.float32)
    p = jnp.exp(e - jnp.max(e))
    jrow = lax.broadcasted_iota(jnp.int32, (_NB, _BR), 0)
    rrow = lax.broadcasted_iota(jnp.int32, (_NB, _BR), 1)
    ohT = (rrow // _W == jrow).astype(jnp.float32)
    ssum = jnp.dot(ohT, p, preferred_element_type=jnp.float32)
    pm = jnp.where(m, p, 0.0)
    ctx = jnp.dot(ohT, X * pm,
                  preferred_element_type=jnp.float32)
    out_ref[...] = ctx / ssum


def _attn_call(bags, idx2, h, W_pre, b_pre2, W_q, W_v, interpret=False):
    grid = (_B * _NN // _NB,)
    return pl.pallas_call(
        _attn_body,
        grid=grid,
        in_specs=[
            pl.BlockSpec((_BR, _DM), lambda s: (s, 0)),
            pl.BlockSpec((_BR, 1), lambda s: (s, 0)),
            pl.BlockSpec((1, 1, _DM), lambda s: (s // (_NN // _NB), 0, 0)),
            pl.BlockSpec((_DM, _DK), lambda s: (0, 0)),
            pl.BlockSpec((1, _DK), lambda s: (0, 0)),
            pl.BlockSpec((_DM, _DK), lambda s: (0, 0)),
            pl.BlockSpec((_DK, 1), lambda s: (0, 0)),
        ],
        out_specs=pl.BlockSpec((_NB, _DM), lambda s: (s, 0)),
        out_shape=jax.ShapeDtypeStruct((_B * _NN, _DM), jnp.float32),
        interpret=interpret,
    )(bags, idx2, h.reshape(_B, 1, _DM), W_pre, b_pre2, W_q, W_v)


def kernel(seq_output, hidden, con_hidden, index, cross_lengths, lengths,
           node_lengths, max_length, emb, W_pre, b_pre, W_q, W_v):
    h = jnp.transpose(con_hidden, (1, 0, 2)).reshape(con_hidden.shape[1], -1)
    idx_flat = index[:, -1].astype(jnp.int32).reshape(_TOT)
    bags = _sc_gather(emb, idx_flat)
    ctx = _attn_call(bags, idx_flat.reshape(_TOT, 1), h, W_pre,
                     b_pre.reshape(1, _DK), W_q, W_v)
    context = jnp.broadcast_to(
        ctx.reshape(_B, 1, _NN, _DM), (_B, _C, _NN, _DM))
    return context, h


# speedup vs baseline: 23.0115x; 1.0388x over previous
import functools

import jax
import jax.numpy as jnp
from jax import lax
from jax.experimental import pallas as pl
from jax.experimental.pallas import tpu as pltpu
from jax.experimental.pallas import tpu_sc as plsc

_B, _C, _NN, _W = 8, 4, 32, 128
_DM, _DK = 512, 64
_PAD = 0

_NC, _NS = 2, 16
_NW = _NC * _NS

_TOT = _B * _NN * _W
_NSPLIT = 2
_HTOT = _TOT // _NSPLIT
_RPW = _HTOT // _NW
_CH = 64
_NBUF = 3
_NCHUNK = _RPW // _CH

_NB = 16
_BR = _NB * _W


def _sc_gather(emb, idx_flat):
    mesh = plsc.VectorSubcoreMesh(core_axis_name="c", subcore_axis_name="s")

    @functools.partial(
        pl.kernel,
        mesh=mesh,
        out_type=jax.ShapeDtypeStruct((_HTOT, _DM), jnp.float32),
        scratch_types=[
            pltpu.VMEM((_RPW,), jnp.int32),
            pltpu.VMEM((_NBUF, _CH, _DM), jnp.float32),
            pltpu.SemaphoreType.DMA,
            pltpu.SemaphoreType.DMA,
        ],
    )
    def gather_k(table_hbm, idx_hbm, out_hbm, idx_v, bufs, gsem, ssem):
        wid = lax.axis_index("s") * _NC + lax.axis_index("c")
        base = wid * _RPW
        pltpu.sync_copy(idx_hbm.at[pl.ds(base, _RPW)], idx_v)
        gathers = [None] * _NCHUNK
        scatters = [None] * _NCHUNK
        for j in range(min(_NBUF - 1, _NCHUNK)):
            gathers[j] = pltpu.async_copy(
                table_hbm.at[idx_v.at[pl.ds(j * _CH, _CH)]],
                bufs.at[j % _NBUF], gsem)
        for i in range(_NCHUNK):
            gathers[i].wait()
            nxt = i + _NBUF - 1
            if nxt < _NCHUNK:
                if nxt >= _NBUF:
                    scatters[nxt - _NBUF].wait()
                gathers[nxt] = pltpu.async_copy(
                    table_hbm.at[idx_v.at[pl.ds(nxt * _CH, _CH)]],
                    bufs.at[nxt % _NBUF], gsem)
            scatters[i] = pltpu.async_copy(
                bufs.at[i % _NBUF], out_hbm.at[pl.ds(base + i * _CH, _CH)],
                ssem)
        for i in range(max(0, _NCHUNK - _NBUF), _NCHUNK):
            if scatters[i] is not None:
                scatters[i].wait()

    return gather_k(emb, idx_flat)


def _attn_body(bags_ref, idx_ref, h_ref, wpre_ref, bpre_ref, wq_ref, wv_ref,
               out_ref):
    X = bags_ref[...]
    m = (idx_ref[...] != _PAD)
    q = jnp.dot(h_ref[0], wq_ref[...],
                preferred_element_type=jnp.float32)
    pre = jnp.where(m, jnp.dot(X, wpre_ref[...],
                               preferred_element_type=jnp.float32), 0.0)
    t = jnp.tanh(pre + bpre_ref[...] + q)
    e = jnp.dot(t, wv_ref[...],
                preferred_element_type=jnp.float32)
    p = jnp.exp(e - jnp.max(e))
    jrow = lax.broadcasted_iota(jnp.int32, (_NB, _BR), 0)
    rrow = lax.broadcasted_iota(jnp.int32, (_NB, _BR), 1)
    ohT = (rrow // _W == jrow).astype(jnp.float32)
    ssum = jnp.dot(ohT, p, preferred_element_type=jnp.float32)
    pm = jnp.where(m, p, 0.0)
    ctx = jnp.dot(ohT, X * pm,
                  preferred_element_type=jnp.float32)
    out_ref[...] = ctx / ssum


def _attn_call(bags, idx2, h_half, W_pre, b_pre2, W_q, W_v, interpret=False):
    nrows = bags.shape[0]
    nb_half = h_half.shape[0]
    grid = (nrows // _BR,)
    return pl.pallas_call(
        _attn_body,
        grid=grid,
        in_specs=[
            pl.BlockSpec((_BR, _DM), lambda s: (s, 0)),
            pl.BlockSpec((_BR, 1), lambda s: (s, 0)),
            pl.BlockSpec((1, 1, _DM), lambda s: (s // (_NN // _NB), 0, 0)),
            pl.BlockSpec((_DM, _DK), lambda s: (0, 0)),
            pl.BlockSpec((1, _DK), lambda s: (0, 0)),
            pl.BlockSpec((_DM, _DK), lambda s: (0, 0)),
            pl.BlockSpec((_DK, 1), lambda s: (0, 0)),
        ],
        out_specs=pl.BlockSpec((_NB, _DM), lambda s: (s, 0)),
        out_shape=jax.ShapeDtypeStruct((nb_half * _NN, _DM), jnp.float32),
        interpret=interpret,
    )(bags, idx2, h_half.reshape(nb_half, 1, _DM), W_pre, b_pre2, W_q, W_v)


def kernel(seq_output, hidden, con_hidden, index, cross_lengths, lengths,
           node_lengths, max_length, emb, W_pre, b_pre, W_q, W_v):
    h = jnp.transpose(con_hidden, (1, 0, 2)).reshape(con_hidden.shape[1], -1)
    idx_flat = index[:, -1].astype(jnp.int32).reshape(_TOT)
    b_pre2 = b_pre.reshape(1, _DK)
    nb_half = _B // _NSPLIT
    ctxs = []
    for s in range(_NSPLIT):
        idx_half = lax.slice(idx_flat, (s * _HTOT,), ((s + 1) * _HTOT,))
        bags = _sc_gather(emb, idx_half)
        ctxs.append(_attn_call(
            bags, idx_half.reshape(_HTOT, 1),
            h[s * nb_half:(s + 1) * nb_half], W_pre, b_pre2, W_q, W_v))
    ctx = jnp.concatenate(ctxs, axis=0)
    context = jnp.broadcast_to(
        ctx.reshape(_B, 1, _NN, _DM), (_B, _C, _NN, _DM))
    return context, h
